# Initial kernel scaffold; baseline (speedup 1.0000x reference)
import jax
import jax.numpy as jnp
from jax.experimental import pallas as pl


def kernel(x, edge_index, edge_attr, W1, b1, W2, b2, Wl, bl):
    raise NotImplementedError("write your pallas kernel here")


# speedup vs baseline: 68.7040x; 68.7040x over previous
import functools

import jax
import jax.numpy as jnp
from jax import lax
from jax.experimental import pallas as pl
from jax.experimental.pallas import tpu as pltpu
from jax.experimental.pallas import tpu_sc as plsc

N = 10000
NP = 10240
D = 128
DH = 64
K = 5
E = 320000
NTILES = 32
CH = 128
NCHUNK = 80
EPAD = NTILES * NCHUNK * CH
NCHUNKC = EPAD // 16 // CH
CHR = 128
STRIPE = NP // 16

_mesh = plsc.VectorSubcoreMesh(core_axis_name="c", subcore_axis_name="s")


def _i32x16(v):
    return jnp.full((16,), v, jnp.int32)


def _z():
    return jnp.int32(0)


def _fori(n, body):
    lax.fori_loop(jnp.int32(0), jnp.int32(n), body, 0)


def _sc_deg_body(rowp, eap, deg_out, row_v, ea_v, buf_v, deg_sh):
    c = lax.axis_index("c")
    s = lax.axis_index("s")
    t = c * 16 + s
    r0 = s * STRIPE
    pltpu.sync_copy(rowp.at[t], row_v)
    pltpu.sync_copy(eap.at[t], ea_v)

    def zb(i, _):
        buf_v[pl.ds(i * 16, 16)] = jnp.zeros((16,), jnp.float32)
        return 0

    _fori(STRIPE // 16, zb)
    pltpu.sync_copy(buf_v, deg_sh.at[pl.ds(r0, STRIPE)])
    plsc.subcore_barrier()

    def chunk(j, _):
        pltpu.sync_copy(ea_v.at[j], deg_sh.at[row_v.at[j]], add=True)
        return 0

    _fori(NCHUNK, chunk)
    plsc.subcore_barrier()
    pltpu.sync_copy(deg_sh.at[pl.ds(r0, STRIPE)], buf_v)
    pltpu.sync_copy(buf_v, deg_out.at[c, pl.ds(r0, STRIPE)])


def _sc_deg(rowp, eap):
    return pl.kernel(
        _sc_deg_body,
        out_type=jax.ShapeDtypeStruct((2, NP), jnp.float32),
        mesh=_mesh,
        scratch_types=[
            pltpu.VMEM((NCHUNK, CH), jnp.int32),
            pltpu.VMEM((NCHUNK, CH), jnp.float32),
            pltpu.VMEM((STRIPE,), jnp.float32),
            pltpu.VMEM_SHARED((NP,), jnp.float32),
        ],
    )(rowp, eap)


def _tc_pro_body(x_ref, degp_ref, dinv_ref, xs_ref):
    deg = degp_ref[0, :] + degp_ref[1, :]
    ok = deg > 0.0
    d = jnp.where(ok, lax.rsqrt(jnp.where(ok, deg, 1.0)), 0.0)
    dinv_ref[0, :] = d
    xs_ref[0] = x_ref[:, :DH]
    xs_ref[1] = x_ref[:, DH:]


def _tc_prologue(xp, deg2):
    br = 1280
    return pl.pallas_call(
        _tc_pro_body,
        grid=(NP // br,),
        in_specs=[
            pl.BlockSpec((br, D), lambda i: (i, _z())),
            pl.BlockSpec((2, br), lambda i: (_z(), i)),
        ],
        out_specs=[
            pl.BlockSpec((1, br), lambda i: (_z(), i)),
            pl.BlockSpec((2, br, DH), lambda i: (_z(), i, _z())),
        ],
        out_shape=[
            jax.ShapeDtypeStruct((1, NP), jnp.float32),
            jax.ShapeDtypeStruct((2, NP, DH), jnp.float32),
        ],
    )(xp, deg2)


G = 8


def _sc_conv_body(xs, dinv, rowp, colp, eap, zz, txs,
                  rg_v, cg_v, eg_v, gb0, gb1, tbuf, pbuf, vbuf, d_v,
                  acc_sh, vin_sh, sem):
    c = lax.axis_index("c")
    s = lax.axis_index("s")
    t = c * 16 + s
    r0 = s * STRIPE

    pltpu.sync_copy(dinv.at[jnp.int32(0), pl.ds(r0, STRIPE)], d_v)
    sb = c * NP
    eb = s * NCHUNKC

    for m in range(STRIPE // CHR):
        rows = pl.ds(r0 + m * CHR, CHR)
        pltpu.sync_copy(xs.at[pl.ds(sb + r0 + m * CHR, CHR), :], tbuf)

        def vrow0(i, _):
            dsp = plsc.load_gather(d_v, [_i32x16(m * CHR + i)])
            for q in range(DH // 16):
                sl = pl.ds(q * 16, 16)
                vbuf[i, sl] = tbuf[i, sl] * dsp
            return 0

        _fori(CHR, vrow0)
        pltpu.sync_copy(vbuf, vin_sh.at[rows, :])
    plsc.subcore_barrier()

    for k in range(1, K):
        sigma = -1.0 if k == 1 else -2.0
        for m in range(STRIPE // CHR):
            pltpu.sync_copy(zz, acc_sh.at[pl.ds(r0 + m * CHR, CHR), :])
        plsc.subcore_barrier()

        def group(g, _):
            gs = pl.ds(eb + g * G, G)
            pltpu.sync_copy(rowp.at[gs, :], rg_v)
            pltpu.sync_copy(colp.at[gs, :], cg_v)
            pltpu.sync_copy(eap.at[gs, :], eg_v)
            for j in range(G):
                gb = gb0 if j % 2 == 0 else gb1
                pltpu.sync_copy(vin_sh.at[rg_v.at[jnp.int32(j)]], gb)

                def erow(i, _, _j=j, _gb=gb):
                    asp = plsc.load_gather(eg_v, [_i32x16(_j), _i32x16(i)])
                    for q in range(DH // 16):
                        sl = pl.ds(q * 16, 16)
                        _gb[i, sl] = _gb[i, sl] * asp
                    return 0

                _fori(CH, erow)
                pltpu.sync_copy(gb, acc_sh.at[cg_v.at[jnp.int32(j)]], add=True)
            return 0

        _fori(NCHUNKC // G, group)
        plsc.subcore_barrier()

        for m in range(STRIPE // CHR):
            rows = pl.ds(r0 + m * CHR, CHR)
            pltpu.sync_copy(acc_sh.at[rows, :], tbuf)
            if k == 2:
                pltpu.sync_copy(xs.at[pl.ds(sb + r0 + m * CHR, CHR), :], pbuf)
            elif k >= 3:
                pltpu.sync_copy(txs.at[jnp.int32(k - 3), pl.ds(sb + r0 + m * CHR, CHR), :], pbuf)

            def crow(i, _):
                dsp = plsc.load_gather(d_v, [_i32x16(m * CHR + i)])
                for q in range(DH // 16):
                    sl = pl.ds(q * 16, 16)
                    tv = (tbuf[i, sl] * dsp) * sigma
                    if k >= 2:
                        tv = tv - pbuf[i, sl]
                    tbuf[i, sl] = tv
                    if k < K - 1:
                        vbuf[i, sl] = tv * dsp
                return 0

            _fori(CHR, crow)
            pltpu.sync_copy(tbuf, txs.at[jnp.int32(k - 1), pl.ds(sb + r0 + m * CHR, CHR), :])
            if k < K - 1:
                pltpu.sync_copy(vbuf, vin_sh.at[rows, :])
        plsc.subcore_barrier()


def _sc_conv(xs, dinv, rowp, colp, eap, zz):
    return pl.kernel(
        _sc_conv_body,
        out_type=jax.ShapeDtypeStruct((K - 1, 2 * NP, DH), jnp.float32),
        mesh=_mesh,
        compiler_params=pltpu.CompilerParams(
            needs_layout_passes=False, use_tc_tiling_on_sc=False),
        scratch_types=[
            pltpu.VMEM((G, CH), jnp.int32),
            pltpu.VMEM((G, CH), jnp.int32),
            pltpu.VMEM((G, CH), jnp.float32),
            pltpu.VMEM((CH, DH), jnp.float32),
            pltpu.VMEM((CH, DH), jnp.float32),
            pltpu.VMEM((CHR, DH), jnp.float32),
            pltpu.VMEM((CHR, DH), jnp.float32),
            pltpu.VMEM((CHR, DH), jnp.float32),
            pltpu.VMEM((STRIPE,), jnp.float32),
            pltpu.VMEM_SHARED((NP, DH), jnp.float32),
            pltpu.VMEM_SHARED((NP, DH), jnp.float32),
            pltpu.SemaphoreType.DMA,
        ],
    )(xs, dinv, rowp, colp, eap, zz)


def _gelu(z):
    return 0.5 * z * (1.0 + lax.erf(z * (2.0 ** -0.5)))


def _cat_body(x, tx_ref):
    parts = [x]
    for k in range(K - 1):
        parts.append(tx_ref[k, 0])
        parts.append(tx_ref[k, 1])
    return jnp.concatenate(parts, axis=1)


def _tc_mid_body(x_ref, tx_ref, w_ref, b_ref, hs_ref):
    cat = _cat_body(x_ref[...], tx_ref)
    y = jnp.dot(cat, w_ref[...], preferred_element_type=jnp.float32)
    h = _gelu(y + b_ref[0, :][None, :])
    hs_ref[0] = h[:, :DH]
    hs_ref[1] = h[:, DH:]


def _tc_mid(xp, txs, wcat, b):
    br = 1024
    return pl.pallas_call(
        _tc_mid_body,
        grid=(NP // br,),
        in_specs=[
            pl.BlockSpec((br, D), lambda i: (i, _z())),
            pl.BlockSpec((K - 1, 2, br, DH), lambda i: (_z(), _z(), i, _z())),
            pl.BlockSpec((K * D, D), lambda i: (_z(), _z())),
            pl.BlockSpec((1, D), lambda i: (_z(), _z())),
        ],
        out_specs=pl.BlockSpec((2, br, DH), lambda i: (_z(), i, _z())),
        out_shape=jax.ShapeDtypeStruct((2, NP, DH), jnp.float32),
    )(xp, txs, wcat, b)


def _tc_fin_body(hs_ref, tx_ref, x_ref, w_ref, b_ref, wlt_ref, bl_ref, o_ref):
    h = jnp.concatenate([hs_ref[0], hs_ref[1]], axis=1)
    cat = _cat_body(h, tx_ref)
    y = jnp.dot(cat, w_ref[...], preferred_element_type=jnp.float32)
    lin = jnp.dot(x_ref[...], wlt_ref[...], preferred_element_type=jnp.float32)
    z = y + b_ref[0, :][None, :] + lin + bl_ref[0, :][None, :]
    o_ref[...] = _gelu(z)


def _tc_fin(hs, txs, xp, wcat, b, wlt, bl):
    br = 1024
    return pl.pallas_call(
        _tc_fin_body,
        grid=(NP // br,),
        in_specs=[
            pl.BlockSpec((2, br, DH), lambda i: (_z(), i, _z())),
            pl.BlockSpec((K - 1, 2, br, DH), lambda i: (_z(), _z(), i, _z())),
            pl.BlockSpec((br, D), lambda i: (i, _z())),
            pl.BlockSpec((K * D, D), lambda i: (_z(), _z())),
            pl.BlockSpec((1, D), lambda i: (_z(), _z())),
            pl.BlockSpec((D, D), lambda i: (_z(), _z())),
            pl.BlockSpec((1, D), lambda i: (_z(), _z())),
        ],
        out_specs=pl.BlockSpec((br, D), lambda i: (i, _z())),
        out_shape=jax.ShapeDtypeStruct((NP, D), jnp.float32),
    )(hs, txs, xp, wcat, b, wlt, bl)


def kernel(x, edge_index, edge_attr, W1, b1, W2, b2, Wl, bl):
    row = edge_index[0].astype(jnp.int32)
    col = edge_index[1].astype(jnp.int32)
    ea = edge_attr.astype(jnp.float32)
    pad = EPAD - E
    rowp = jnp.concatenate([row, jnp.zeros((pad,), jnp.int32)]).reshape(
        NTILES, NCHUNK, CH)
    colp = jnp.concatenate([col, jnp.zeros((pad,), jnp.int32)]).reshape(
        NTILES, NCHUNK, CH)
    eap = jnp.concatenate([ea, jnp.zeros((pad,), jnp.float32)]).reshape(
        NTILES, NCHUNK, CH)
    xp = jnp.pad(x.astype(jnp.float32), ((0, NP - N), (0, 0)))

    wcat1 = W1.astype(jnp.float32).reshape(K * D, D)
    wcat2 = W2.astype(jnp.float32).reshape(K * D, D)
    wlt = Wl.astype(jnp.float32).T
    b1r = b1.astype(jnp.float32).reshape(1, D)
    b2r = b2.astype(jnp.float32).reshape(1, D)
    blr = bl.astype(jnp.float32).reshape(1, D)

    rowf = rowp.reshape(NTILES * NCHUNK, CH)
    colf = colp.reshape(NTILES * NCHUNK, CH)
    eaf = eap.reshape(NTILES * NCHUNK, CH)
    deg2 = _sc_deg(rowp, eap)
    dinv, xs = _tc_prologue(xp, deg2)
    xsf = xs.reshape(2 * NP, DH)
    zz = jnp.zeros((CHR, DH), jnp.float32)
    txs1 = _sc_conv(xsf, dinv, rowf, colf, eaf, zz).reshape(K - 1, 2, NP, DH)
    hs = _tc_mid(xp, txs1, wcat1, b1r)
    txs2 = _sc_conv(hs.reshape(2 * NP, DH), dinv, rowf, colf, eaf, zz).reshape(K - 1, 2, NP, DH)
    out = _tc_fin(hs, txs2, xp, wcat2, b2r, wlt, blr)
    return out[:N].astype(jnp.float64)


# speedup vs baseline: 84.9464x; 1.2364x over previous
import functools

import jax
import jax.numpy as jnp
from jax import lax
from jax.experimental import pallas as pl
from jax.experimental.pallas import tpu as pltpu
from jax.experimental.pallas import tpu_sc as plsc

N = 10000
NP = 10240
D = 128
DH = 64
K = 5
E = 320000
NTILES = 32
CH = 128
NCHUNK = 80
EPAD = NTILES * NCHUNK * CH
NCHUNKC = EPAD // 16 // CH
CHR = 128
STRIPE = NP // 16

_mesh = plsc.VectorSubcoreMesh(core_axis_name="c", subcore_axis_name="s")


def _i32x16(v):
    return jnp.full((16,), v, jnp.int32)


def _z():
    return jnp.int32(0)


def _fori(n, body):
    lax.fori_loop(jnp.int32(0), jnp.int32(n), body, 0)


def _sc_deg_body(rowp, eap, deg_out, row_v, ea_v, buf_v, deg_sh):
    c = lax.axis_index("c")
    s = lax.axis_index("s")
    t = c * 16 + s
    r0 = s * STRIPE
    pltpu.sync_copy(rowp.at[t], row_v)
    pltpu.sync_copy(eap.at[t], ea_v)

    def zb(i, _):
        buf_v[pl.ds(i * 16, 16)] = jnp.zeros((16,), jnp.float32)
        return 0

    _fori(STRIPE // 16, zb)
    pltpu.sync_copy(buf_v, deg_sh.at[pl.ds(r0, STRIPE)])
    plsc.subcore_barrier()

    def chunk(j, _):
        pltpu.sync_copy(ea_v.at[j], deg_sh.at[row_v.at[j]], add=True)
        return 0

    _fori(NCHUNK, chunk)
    plsc.subcore_barrier()
    pltpu.sync_copy(deg_sh.at[pl.ds(r0, STRIPE)], buf_v)
    pltpu.sync_copy(buf_v, deg_out.at[c, pl.ds(r0, STRIPE)])


def _sc_deg(rowp, eap):
    return pl.kernel(
        _sc_deg_body,
        out_type=jax.ShapeDtypeStruct((2, NP), jnp.float32),
        mesh=_mesh,
        scratch_types=[
            pltpu.VMEM((NCHUNK, CH), jnp.int32),
            pltpu.VMEM((NCHUNK, CH), jnp.float32),
            pltpu.VMEM((STRIPE,), jnp.float32),
            pltpu.VMEM_SHARED((NP,), jnp.float32),
        ],
    )(rowp, eap)


def _tc_pro_body(x_ref, degp_ref, dinv_ref, xs_ref):
    deg = degp_ref[0, :] + degp_ref[1, :]
    ok = deg > 0.0
    d = jnp.where(ok, lax.rsqrt(jnp.where(ok, deg, 1.0)), 0.0)
    dinv_ref[0, :] = d
    xs_ref[0] = x_ref[:, :DH]
    xs_ref[1] = x_ref[:, DH:]


def _tc_prologue(xp, deg2):
    br = 1280
    return pl.pallas_call(
        _tc_pro_body,
        grid=(NP // br,),
        in_specs=[
            pl.BlockSpec((br, D), lambda i: (i, _z())),
            pl.BlockSpec((2, br), lambda i: (_z(), i)),
        ],
        out_specs=[
            pl.BlockSpec((1, br), lambda i: (_z(), i)),
            pl.BlockSpec((2, br, DH), lambda i: (_z(), i, _z())),
        ],
        out_shape=[
            jax.ShapeDtypeStruct((1, NP), jnp.float32),
            jax.ShapeDtypeStruct((2, NP, DH), jnp.float32),
        ],
    )(xp, deg2)


G = 8


def _sc_conv_body(xs, dinv, rowp, colp, eap, zz, txs,
                  rg_v, cg_v, eg_v, gb0, gb1, tbuf, pbuf, vbuf, d_v,
                  acc_sh, vin_sh, sem):
    c = lax.axis_index("c")
    s = lax.axis_index("s")
    t = c * 16 + s
    r0 = s * STRIPE

    pltpu.sync_copy(dinv.at[jnp.int32(0), pl.ds(r0, STRIPE)], d_v)
    sb = c * NP
    eb = s * NCHUNKC

    for m in range(STRIPE // CHR):
        rows = pl.ds(r0 + m * CHR, CHR)
        pltpu.sync_copy(xs.at[pl.ds(sb + r0 + m * CHR, CHR), :], tbuf)

        def vrow0(i, _):
            dsp = plsc.load_gather(d_v, [_i32x16(m * CHR + i)])
            for q in range(DH // 16):
                sl = pl.ds(q * 16, 16)
                vbuf[i, sl] = tbuf[i, sl] * dsp
            return 0

        _fori(CHR, vrow0)
        pltpu.sync_copy(vbuf, vin_sh.at[rows, :])
    plsc.subcore_barrier()

    for k in range(1, K):
        sigma = -1.0 if k == 1 else -2.0
        for m in range(STRIPE // CHR):
            pltpu.sync_copy(zz, acc_sh.at[pl.ds(r0 + m * CHR, CHR), :])
        plsc.subcore_barrier()

        def group(g, _):
            gs = pl.ds(eb + g * G, G)
            pltpu.sync_copy(rowp.at[gs, :], rg_v)
            pltpu.sync_copy(colp.at[gs, :], cg_v)
            pltpu.sync_copy(eap.at[gs, :], eg_v)
            pltpu.async_copy(vin_sh.at[rg_v.at[jnp.int32(0)]], gb0, sem)
            for j in range(G):
                gb = gb0 if j % 2 == 0 else gb1
                pltpu.make_async_copy(
                    vin_sh.at[rg_v.at[jnp.int32(j)]], gb, sem).wait()
                if j + 1 < G:
                    nb = gb1 if j % 2 == 0 else gb0
                    pltpu.async_copy(
                        vin_sh.at[rg_v.at[jnp.int32(j + 1)]], nb, sem)

                def erow4(i, _, _j=j, _gb=gb):
                    base = i * 4
                    for u in range(4):
                        iv = base + u
                        asp = plsc.load_gather(
                            eg_v, [_i32x16(_j), _i32x16(iv)])
                        for q in range(DH // 16):
                            sl = pl.ds(q * 16, 16)
                            _gb[iv, sl] = _gb[iv, sl] * asp
                    return 0

                _fori(CH // 4, erow4)
                pltpu.sync_copy(gb, acc_sh.at[cg_v.at[jnp.int32(j)]], add=True)
            return 0

        _fori(NCHUNKC // G, group)
        plsc.subcore_barrier()

        for m in range(STRIPE // CHR):
            rows = pl.ds(r0 + m * CHR, CHR)
            pltpu.sync_copy(acc_sh.at[rows, :], tbuf)
            if k == 2:
                pltpu.sync_copy(xs.at[pl.ds(sb + r0 + m * CHR, CHR), :], pbuf)
            elif k >= 3:
                pltpu.sync_copy(txs.at[jnp.int32(k - 3), pl.ds(sb + r0 + m * CHR, CHR), :], pbuf)

            def crow(i, _):
                dsp = plsc.load_gather(d_v, [_i32x16(m * CHR + i)])
                for q in range(DH // 16):
                    sl = pl.ds(q * 16, 16)
                    tv = (tbuf[i, sl] * dsp) * sigma
                    if k >= 2:
                        tv = tv - pbuf[i, sl]
                    tbuf[i, sl] = tv
                    if k < K - 1:
                        vbuf[i, sl] = tv * dsp
                return 0

            _fori(CHR, crow)
            pltpu.sync_copy(tbuf, txs.at[jnp.int32(k - 1), pl.ds(sb + r0 + m * CHR, CHR), :])
            if k < K - 1:
                pltpu.sync_copy(vbuf, vin_sh.at[rows, :])
        plsc.subcore_barrier()


def _sc_conv(xs, dinv, rowp, colp, eap, zz):
    return pl.kernel(
        _sc_conv_body,
        out_type=jax.ShapeDtypeStruct((K - 1, 2 * NP, DH), jnp.float32),
        mesh=_mesh,
        compiler_params=pltpu.CompilerParams(
            needs_layout_passes=False, use_tc_tiling_on_sc=False),
        scratch_types=[
            pltpu.VMEM((G, CH), jnp.int32),
            pltpu.VMEM((G, CH), jnp.int32),
            pltpu.VMEM((G, CH), jnp.float32),
            pltpu.VMEM((CH, DH), jnp.float32),
            pltpu.VMEM((CH, DH), jnp.float32),
            pltpu.VMEM((CHR, DH), jnp.float32),
            pltpu.VMEM((CHR, DH), jnp.float32),
            pltpu.VMEM((CHR, DH), jnp.float32),
            pltpu.VMEM((STRIPE,), jnp.float32),
            pltpu.VMEM_SHARED((NP, DH), jnp.float32),
            pltpu.VMEM_SHARED((NP, DH), jnp.float32),
            pltpu.SemaphoreType.DMA,
        ],
    )(xs, dinv, rowp, colp, eap, zz)


def _gelu(z):
    return 0.5 * z * (1.0 + lax.erf(z * (2.0 ** -0.5)))


def _cat_body(x, tx_ref):
    parts = [x]
    for k in range(K - 1):
        parts.append(tx_ref[k, 0])
        parts.append(tx_ref[k, 1])
    return jnp.concatenate(parts, axis=1)


def _tc_mid_body(x_ref, tx_ref, w_ref, b_ref, hs_ref):
    cat = _cat_body(x_ref[...], tx_ref)
    y = jnp.dot(cat, w_ref[...], preferred_element_type=jnp.float32)
    h = _gelu(y + b_ref[0, :][None, :])
    hs_ref[0] = h[:, :DH]
    hs_ref[1] = h[:, DH:]


def _tc_mid(xp, txs, wcat, b):
    br = 1024
    return pl.pallas_call(
        _tc_mid_body,
        grid=(NP // br,),
        in_specs=[
            pl.BlockSpec((br, D), lambda i: (i, _z())),
            pl.BlockSpec((K - 1, 2, br, DH), lambda i: (_z(), _z(), i, _z())),
            pl.BlockSpec((K * D, D), lambda i: (_z(), _z())),
            pl.BlockSpec((1, D), lambda i: (_z(), _z())),
        ],
        out_specs=pl.BlockSpec((2, br, DH), lambda i: (_z(), i, _z())),
        out_shape=jax.ShapeDtypeStruct((2, NP, DH), jnp.float32),
    )(xp, txs, wcat, b)


def _tc_fin_body(hs_ref, tx_ref, x_ref, w_ref, b_ref, wlt_ref, bl_ref, o_ref):
    h = jnp.concatenate([hs_ref[0], hs_ref[1]], axis=1)
    cat = _cat_body(h, tx_ref)
    y = jnp.dot(cat, w_ref[...], preferred_element_type=jnp.float32)
    lin = jnp.dot(x_ref[...], wlt_ref[...], preferred_element_type=jnp.float32)
    z = y + b_ref[0, :][None, :] + lin + bl_ref[0, :][None, :]
    o_ref[...] = _gelu(z)


def _tc_fin(hs, txs, xp, wcat, b, wlt, bl):
    br = 1024
    return pl.pallas_call(
        _tc_fin_body,
        grid=(NP // br,),
        in_specs=[
            pl.BlockSpec((2, br, DH), lambda i: (_z(), i, _z())),
            pl.BlockSpec((K - 1, 2, br, DH), lambda i: (_z(), _z(), i, _z())),
            pl.BlockSpec((br, D), lambda i: (i, _z())),
            pl.BlockSpec((K * D, D), lambda i: (_z(), _z())),
            pl.BlockSpec((1, D), lambda i: (_z(), _z())),
            pl.BlockSpec((D, D), lambda i: (_z(), _z())),
            pl.BlockSpec((1, D), lambda i: (_z(), _z())),
        ],
        out_specs=pl.BlockSpec((br, D), lambda i: (i, _z())),
        out_shape=jax.ShapeDtypeStruct((NP, D), jnp.float32),
    )(hs, txs, xp, wcat, b, wlt, bl)


def kernel(x, edge_index, edge_attr, W1, b1, W2, b2, Wl, bl):
    row = edge_index[0].astype(jnp.int32)
    col = edge_index[1].astype(jnp.int32)
    ea = edge_attr.astype(jnp.float32)
    pad = EPAD - E
    rowp = jnp.concatenate([row, jnp.zeros((pad,), jnp.int32)]).reshape(
        NTILES, NCHUNK, CH)
    colp = jnp.concatenate([col, jnp.zeros((pad,), jnp.int32)]).reshape(
        NTILES, NCHUNK, CH)
    eap = jnp.concatenate([ea, jnp.zeros((pad,), jnp.float32)]).reshape(
        NTILES, NCHUNK, CH)
    xp = jnp.pad(x.astype(jnp.float32), ((0, NP - N), (0, 0)))

    wcat1 = W1.astype(jnp.float32).reshape(K * D, D)
    wcat2 = W2.astype(jnp.float32).reshape(K * D, D)
    wlt = Wl.astype(jnp.float32).T
    b1r = b1.astype(jnp.float32).reshape(1, D)
    b2r = b2.astype(jnp.float32).reshape(1, D)
    blr = bl.astype(jnp.float32).reshape(1, D)

    rowf = rowp.reshape(NTILES * NCHUNK, CH)
    colf = colp.reshape(NTILES * NCHUNK, CH)
    eaf = eap.reshape(NTILES * NCHUNK, CH)
    deg2 = _sc_deg(rowp, eap)
    dinv, xs = _tc_prologue(xp, deg2)
    xsf = xs.reshape(2 * NP, DH)
    zz = jnp.zeros((CHR, DH), jnp.float32)
    txs1 = _sc_conv(xsf, dinv, rowf, colf, eaf, zz).reshape(K - 1, 2, NP, DH)
    hs = _tc_mid(xp, txs1, wcat1, b1r)
    txs2 = _sc_conv(hs.reshape(2 * NP, DH), dinv, rowf, colf, eaf, zz).reshape(K - 1, 2, NP, DH)
    out = _tc_fin(hs, txs2, xp, wcat2, b2r, wlt, blr)
    return out[:N].astype(jnp.float64)


# speedup vs baseline: 91.7570x; 1.0802x over previous
import functools

import jax
import jax.numpy as jnp
from jax import lax
from jax.experimental import pallas as pl
from jax.experimental.pallas import tpu as pltpu
from jax.experimental.pallas import tpu_sc as plsc

N = 10000
NP = 10240
D = 128
DH = 64
K = 5
E = 320000
NTILES = 32
CH = 128
NCHUNK = 80
EPAD = NTILES * NCHUNK * CH
NCHUNKC = EPAD // 16 // CH
CHR = 128
STRIPE = NP // 16

_mesh = plsc.VectorSubcoreMesh(core_axis_name="c", subcore_axis_name="s")


def _i32x16(v):
    return jnp.full((16,), v, jnp.int32)


def _z():
    return jnp.int32(0)


def _fori(n, body):
    lax.fori_loop(jnp.int32(0), jnp.int32(n), body, 0)


def _sc_deg_body(rowp, eap, deg_out, row_v, ea_v, buf_v, deg_sh):
    c = lax.axis_index("c")
    s = lax.axis_index("s")
    t = c * 16 + s
    r0 = s * STRIPE
    pltpu.sync_copy(rowp.at[t], row_v)
    pltpu.sync_copy(eap.at[t], ea_v)

    def zb(i, _):
        buf_v[pl.ds(i * 16, 16)] = jnp.zeros((16,), jnp.float32)
        return 0

    _fori(STRIPE // 16, zb)
    pltpu.sync_copy(buf_v, deg_sh.at[pl.ds(r0, STRIPE)])
    plsc.subcore_barrier()

    def chunk(j, _):
        pltpu.sync_copy(ea_v.at[j], deg_sh.at[row_v.at[j]], add=True)
        return 0

    _fori(NCHUNK, chunk)
    plsc.subcore_barrier()
    pltpu.sync_copy(deg_sh.at[pl.ds(r0, STRIPE)], buf_v)
    pltpu.sync_copy(buf_v, deg_out.at[c, pl.ds(r0, STRIPE)])


def _sc_deg(rowp, eap):
    return pl.kernel(
        _sc_deg_body,
        out_type=jax.ShapeDtypeStruct((2, NP), jnp.float32),
        mesh=_mesh,
        scratch_types=[
            pltpu.VMEM((NCHUNK, CH), jnp.int32),
            pltpu.VMEM((NCHUNK, CH), jnp.float32),
            pltpu.VMEM((STRIPE,), jnp.float32),
            pltpu.VMEM_SHARED((NP,), jnp.float32),
        ],
    )(rowp, eap)


def _tc_pro_body(x_ref, degp_ref, dinv_ref, xs_ref):
    deg = degp_ref[0, :] + degp_ref[1, :]
    ok = deg > 0.0
    d = jnp.where(ok, lax.rsqrt(jnp.where(ok, deg, 1.0)), 0.0)
    dinv_ref[0, :] = d
    xs_ref[0] = x_ref[:, :DH]
    xs_ref[1] = x_ref[:, DH:]


def _tc_prologue(xp, deg2):
    br = 1280
    return pl.pallas_call(
        _tc_pro_body,
        grid=(NP // br,),
        in_specs=[
            pl.BlockSpec((br, D), lambda i: (i, _z())),
            pl.BlockSpec((2, br), lambda i: (_z(), i)),
        ],
        out_specs=[
            pl.BlockSpec((1, br), lambda i: (_z(), i)),
            pl.BlockSpec((2, br, DH), lambda i: (_z(), i, _z())),
        ],
        out_shape=[
            jax.ShapeDtypeStruct((1, NP), jnp.float32),
            jax.ShapeDtypeStruct((2, NP, DH), jnp.float32),
        ],
    )(xp, deg2)


G = 8


def _sc_conv_body(xs, dinv, rowp, colp, eap, zz, txs,
                  rg_v, cg_v, eg_v, gb0, gb1, tbuf, pbuf, vbuf, d_v,
                  acc_sh, vin_sh, sem, sem2):
    c = lax.axis_index("c")
    s = lax.axis_index("s")
    t = c * 16 + s
    r0 = s * STRIPE

    pltpu.sync_copy(dinv.at[jnp.int32(0), pl.ds(r0, STRIPE)], d_v)
    sb = c * NP
    eb = s * NCHUNKC

    for m in range(STRIPE // CHR):
        rows = pl.ds(r0 + m * CHR, CHR)
        pltpu.sync_copy(xs.at[pl.ds(sb + r0 + m * CHR, CHR), :], tbuf)

        def vrow0(i, _):
            dsp = plsc.load_gather(d_v, [_i32x16(m * CHR + i)])
            for q in range(DH // 16):
                sl = pl.ds(q * 16, 16)
                vbuf[i, sl] = tbuf[i, sl] * dsp
            return 0

        _fori(CHR, vrow0)
        pltpu.sync_copy(vbuf, vin_sh.at[rows, :])
    plsc.subcore_barrier()

    for k in range(1, K):
        sigma = -1.0 if k == 1 else -2.0
        for m in range(STRIPE // CHR):
            pltpu.sync_copy(zz, acc_sh.at[pl.ds(r0 + m * CHR, CHR), :])
        plsc.subcore_barrier()

        def group(g, _):
            gs = pl.ds(eb + g * G, G)
            c1 = pltpu.async_copy(rowp.at[gs, :], rg_v, sem2)
            c2 = pltpu.async_copy(colp.at[gs, :], cg_v, sem2)
            c3 = pltpu.async_copy(eap.at[gs, :], eg_v, sem2)
            c1.wait()
            c2.wait()
            c3.wait()
            pltpu.async_copy(vin_sh.at[rg_v.at[jnp.int32(0)]], gb0, sem)
            for j in range(G):
                gb = gb0 if j % 2 == 0 else gb1
                pltpu.make_async_copy(
                    vin_sh.at[rg_v.at[jnp.int32(j)]], gb, sem).wait()
                if j + 1 < G:
                    nb = gb1 if j % 2 == 0 else gb0
                    pltpu.async_copy(
                        vin_sh.at[rg_v.at[jnp.int32(j + 1)]], nb, sem)

                def erow4(i, _, _j=j, _gb=gb):
                    base = i * 4
                    for u in range(4):
                        iv = base + u
                        asp = plsc.load_gather(
                            eg_v, [_i32x16(_j), _i32x16(iv)])
                        for q in range(DH // 16):
                            sl = pl.ds(q * 16, 16)
                            _gb[iv, sl] = _gb[iv, sl] * asp
                    return 0

                _fori(CH // 4, erow4)
                pltpu.sync_copy(gb, acc_sh.at[cg_v.at[jnp.int32(j)]], add=True)
            return 0

        _fori(NCHUNKC // G, group)
        plsc.subcore_barrier()

        for m in range(STRIPE // CHR):
            rows = pl.ds(r0 + m * CHR, CHR)
            pltpu.sync_copy(acc_sh.at[rows, :], tbuf)
            if k == 2:
                pltpu.sync_copy(xs.at[pl.ds(sb + r0 + m * CHR, CHR), :], pbuf)
            elif k >= 3:
                pltpu.sync_copy(txs.at[jnp.int32(k - 3), pl.ds(sb + r0 + m * CHR, CHR), :], pbuf)

            def crow(i, _):
                dsp = plsc.load_gather(d_v, [_i32x16(m * CHR + i)])
                for q in range(DH // 16):
                    sl = pl.ds(q * 16, 16)
                    tv = (tbuf[i, sl] * dsp) * sigma
                    if k >= 2:
                        tv = tv - pbuf[i, sl]
                    tbuf[i, sl] = tv
                    if k < K - 1:
                        vbuf[i, sl] = tv * dsp
                return 0

            _fori(CHR, crow)
            pltpu.sync_copy(tbuf, txs.at[jnp.int32(k - 1), pl.ds(sb + r0 + m * CHR, CHR), :])
            if k < K - 1:
                pltpu.sync_copy(vbuf, vin_sh.at[rows, :])
        plsc.subcore_barrier()


def _sc_conv(xs, dinv, rowp, colp, eap, zz):
    return pl.kernel(
        _sc_conv_body,
        out_type=jax.ShapeDtypeStruct((K - 1, 2 * NP, DH), jnp.float32),
        mesh=_mesh,
        compiler_params=pltpu.CompilerParams(
            needs_layout_passes=False, use_tc_tiling_on_sc=False),
        scratch_types=[
            pltpu.VMEM((G, CH), jnp.int32),
            pltpu.VMEM((G, CH), jnp.int32),
            pltpu.VMEM((G, CH), jnp.float32),
            pltpu.VMEM((CH, DH), jnp.float32),
            pltpu.VMEM((CH, DH), jnp.float32),
            pltpu.VMEM((CHR, DH), jnp.float32),
            pltpu.VMEM((CHR, DH), jnp.float32),
            pltpu.VMEM((CHR, DH), jnp.float32),
            pltpu.VMEM((STRIPE,), jnp.float32),
            pltpu.VMEM_SHARED((NP, DH), jnp.float32),
            pltpu.VMEM_SHARED((NP, DH), jnp.float32),
            pltpu.SemaphoreType.DMA,
            pltpu.SemaphoreType.DMA,
        ],
    )(xs, dinv, rowp, colp, eap, zz)


def _gelu(z):
    return 0.5 * z * (1.0 + lax.erf(z * (2.0 ** -0.5)))


def _cat_body(x, tx_ref):
    parts = [x]
    for k in range(K - 1):
        parts.append(tx_ref[k, 0])
        parts.append(tx_ref[k, 1])
    return jnp.concatenate(parts, axis=1)


def _tc_mid_body(x_ref, tx_ref, w_ref, b_ref, hs_ref):
    cat = _cat_body(x_ref[...], tx_ref)
    y = jnp.dot(cat, w_ref[...], preferred_element_type=jnp.float32)
    h = _gelu(y + b_ref[0, :][None, :])
    hs_ref[0] = h[:, :DH]
    hs_ref[1] = h[:, DH:]


def _tc_mid(xp, txs, wcat, b):
    br = 1024
    return pl.pallas_call(
        _tc_mid_body,
        grid=(NP // br,),
        in_specs=[
            pl.BlockSpec((br, D), lambda i: (i, _z())),
            pl.BlockSpec((K - 1, 2, br, DH), lambda i: (_z(), _z(), i, _z())),
            pl.BlockSpec((K * D, D), lambda i: (_z(), _z())),
            pl.BlockSpec((1, D), lambda i: (_z(), _z())),
        ],
        out_specs=pl.BlockSpec((2, br, DH), lambda i: (_z(), i, _z())),
        out_shape=jax.ShapeDtypeStruct((2, NP, DH), jnp.float32),
    )(xp, txs, wcat, b)


def _tc_fin_body(hs_ref, tx_ref, x_ref, w_ref, b_ref, wlt_ref, bl_ref, o_ref):
    h = jnp.concatenate([hs_ref[0], hs_ref[1]], axis=1)
    cat = _cat_body(h, tx_ref)
    y = jnp.dot(cat, w_ref[...], preferred_element_type=jnp.float32)
    lin = jnp.dot(x_ref[...], wlt_ref[...], preferred_element_type=jnp.float32)
    z = y + b_ref[0, :][None, :] + lin + bl_ref[0, :][None, :]
    o_ref[...] = _gelu(z)


def _tc_fin(hs, txs, xp, wcat, b, wlt, bl):
    br = 1024
    return pl.pallas_call(
        _tc_fin_body,
        grid=(NP // br,),
        in_specs=[
            pl.BlockSpec((2, br, DH), lambda i: (_z(), i, _z())),
            pl.BlockSpec((K - 1, 2, br, DH), lambda i: (_z(), _z(), i, _z())),
            pl.BlockSpec((br, D), lambda i: (i, _z())),
            pl.BlockSpec((K * D, D), lambda i: (_z(), _z())),
            pl.BlockSpec((1, D), lambda i: (_z(), _z())),
            pl.BlockSpec((D, D), lambda i: (_z(), _z())),
            pl.BlockSpec((1, D), lambda i: (_z(), _z())),
        ],
        out_specs=pl.BlockSpec((br, D), lambda i: (i, _z())),
        out_shape=jax.ShapeDtypeStruct((NP, D), jnp.float32),
    )(hs, txs, xp, wcat, b, wlt, bl)


def kernel(x, edge_index, edge_attr, W1, b1, W2, b2, Wl, bl):
    row = edge_index[0].astype(jnp.int32)
    col = edge_index[1].astype(jnp.int32)
    ea = edge_attr.astype(jnp.float32)
    pad = EPAD - E
    rowp = jnp.concatenate([row, jnp.zeros((pad,), jnp.int32)]).reshape(
        NTILES, NCHUNK, CH)
    colp = jnp.concatenate([col, jnp.zeros((pad,), jnp.int32)]).reshape(
        NTILES, NCHUNK, CH)
    eap = jnp.concatenate([ea, jnp.zeros((pad,), jnp.float32)]).reshape(
        NTILES, NCHUNK, CH)
    xp = jnp.pad(x.astype(jnp.float32), ((0, NP - N), (0, 0)))

    wcat1 = W1.astype(jnp.float32).reshape(K * D, D)
    wcat2 = W2.astype(jnp.float32).reshape(K * D, D)
    wlt = Wl.astype(jnp.float32).T
    b1r = b1.astype(jnp.float32).reshape(1, D)
    b2r = b2.astype(jnp.float32).reshape(1, D)
    blr = bl.astype(jnp.float32).reshape(1, D)

    rowf = rowp.reshape(NTILES * NCHUNK, CH)
    colf = colp.reshape(NTILES * NCHUNK, CH)
    eaf = eap.reshape(NTILES * NCHUNK, CH)
    deg2 = _sc_deg(rowp, eap)
    dinv, xs = _tc_prologue(xp, deg2)
    xsf = xs.reshape(2 * NP, DH)
    zz = jnp.zeros((CHR, DH), jnp.float32)
    txs1 = _sc_conv(xsf, dinv, rowf, colf, eaf, zz).reshape(K - 1, 2, NP, DH)
    hs = _tc_mid(xp, txs1, wcat1, b1r)
    txs2 = _sc_conv(hs.reshape(2 * NP, DH), dinv, rowf, colf, eaf, zz).reshape(K - 1, 2, NP, DH)
    out = _tc_fin(hs, txs2, xp, wcat2, b2r, wlt, blr)
    return out[:N].astype(jnp.float64)
